# Initial kernel scaffold; baseline (speedup 1.0000x reference)
import functools

import jax
import jax.numpy as jnp
from jax import lax
from jax.experimental import pallas as pl
from jax.experimental.pallas import tpu as pltpu
from jax.experimental.pallas import tpu_sc as plsc

N_NODES = 10000
D = 128
E_BLOCK = 320000
E_PRED = 160000

NC = 2
NS = 16
NW = NC * NS
CHUNK = 128
ROWS_PER_TILE = N_NODES // NS
DEGW = 16

E_CHUNKS = E_BLOCK // CHUNK
P_CHUNKS = E_PRED // CHUNK

_mesh = plsc.VectorSubcoreMesh(core_axis_name="c", subcore_axis_name="s")


@functools.partial(
    pl.kernel,
    mesh=_mesh,
    out_type=[
        jax.ShapeDtypeStruct((NC, N_NODES, D), jnp.float32),
        jax.ShapeDtypeStruct((NC, N_NODES, DEGW), jnp.float32),
    ],
    scratch_types=[
        pltpu.VMEM_SHARED((N_NODES, D), jnp.float32),
        pltpu.VMEM_SHARED((N_NODES, DEGW), jnp.float32),
        pltpu.VMEM((CHUNK,), jnp.int32),
        pltpu.VMEM((CHUNK,), jnp.int32),
        pltpu.VMEM((CHUNK, D), jnp.float32),
        pltpu.VMEM((CHUNK, DEGW), jnp.float32),
        pltpu.SemaphoreType.DMA,
    ],
)
def _aggregate(feats_h, src_h, dst_h, zrow_h, zdeg_h, ones_h,
               aggp_h, degp_h,
               agg_s, deg_s, sidx_v, didx_v, rows_v, ones_v, sem):
    c = lax.axis_index("c")
    s = lax.axis_index("s")
    wid = s * NC + c

    r0 = s * ROWS_PER_TILE
    pltpu.sync_copy(zrow_h, agg_s.at[pl.ds(r0, ROWS_PER_TILE)])
    pltpu.sync_copy(zdeg_h, deg_s.at[pl.ds(r0, ROWS_PER_TILE)])
    pltpu.sync_copy(ones_h, ones_v)
    plsc.subcore_barrier()

    nj = 78 + jnp.where(wid < E_CHUNKS - 78 * NW, 1, 0)

    def body(j, carry):
        base = (wid + j * NW) * CHUNK
        pltpu.sync_copy(src_h.at[pl.ds(base, CHUNK)], sidx_v)
        pltpu.sync_copy(dst_h.at[pl.ds(base, CHUNK)], didx_v)
        pltpu.async_copy(feats_h.at[sidx_v], rows_v, sem).wait()
        pltpu.sync_copy(rows_v, agg_s.at[didx_v], add=True)
        pltpu.sync_copy(ones_v, deg_s.at[didx_v], add=True)
        return carry

    lax.fori_loop(0, nj, body, 0)
    plsc.subcore_barrier()

    pltpu.sync_copy(agg_s.at[pl.ds(r0, ROWS_PER_TILE)],
                    aggp_h.at[c, pl.ds(r0, ROWS_PER_TILE)])
    pltpu.sync_copy(deg_s.at[pl.ds(r0, ROWS_PER_TILE)],
                    degp_h.at[c, pl.ds(r0, ROWS_PER_TILE)])


_BR = 1000


def _encode_block(aggp_ref, degp_ref, feats_ref, w_ref, wself_ref, out_ref):
    a = aggp_ref[0] + aggp_ref[1]
    degm = degp_ref[0] + degp_ref[1]
    deg = jnp.maximum(degm[:, 0:1], 1.0)
    norm = a / deg
    h = (jnp.dot(norm, w_ref[...], preferred_element_type=jnp.float32)
         + jnp.dot(feats_ref[...], wself_ref[...],
                   preferred_element_type=jnp.float32))
    out_ref[...] = jnp.maximum(h, 0.0)


def _encode(aggp, degp, feats, W, W_self):
    grid = N_NODES // _BR
    return pl.pallas_call(
        _encode_block,
        grid=(grid,),
        in_specs=[
            pl.BlockSpec((NC, _BR, D), lambda i: (0, i, 0)),
            pl.BlockSpec((NC, _BR, DEGW), lambda i: (0, i, 0)),
            pl.BlockSpec((_BR, D), lambda i: (i, 0)),
            pl.BlockSpec((D, D), lambda i: (0, 0)),
            pl.BlockSpec((D, D), lambda i: (0, 0)),
        ],
        out_specs=pl.BlockSpec((_BR, D), lambda i: (i, 0)),
        out_shape=jax.ShapeDtypeStruct((N_NODES, D), jnp.float32),
    )(aggp, degp, feats, W, W_self)


@functools.partial(
    pl.kernel,
    mesh=_mesh,
    out_type=[
        jax.ShapeDtypeStruct((E_PRED,), jnp.float32),
        jax.ShapeDtypeStruct((E_PRED,), jnp.float32),
    ],
    scratch_types=[
        pltpu.VMEM_SHARED((N_NODES, D), jnp.float32),
        pltpu.VMEM((CHUNK,), jnp.int32),
        pltpu.VMEM((CHUNK,), jnp.int32),
        pltpu.VMEM((CHUNK, D), jnp.float32),
        pltpu.VMEM((CHUNK, D), jnp.float32),
        pltpu.VMEM((CHUNK,), jnp.float32),
        pltpu.SemaphoreType.DMA,
    ],
)
def _score(h_h, pu_h, pv_h, nu_h, nv_h,
           pos_h, neg_h,
           h_s, uidx_v, vidx_v, hu_v, hv_v, scr_v, sem):
    c = lax.axis_index("c")
    s = lax.axis_index("s")
    wid = s * NC + c

    r0 = s * ROWS_PER_TILE
    pltpu.sync_copy(h_h.at[pl.ds(r0, ROWS_PER_TILE)],
                    h_s.at[pl.ds(r0, ROWS_PER_TILE)])
    plsc.subcore_barrier()

    def chunk_scores(u_h, v_h, out_h, base):
        pltpu.sync_copy(u_h.at[pl.ds(base, CHUNK)], uidx_v)
        pltpu.sync_copy(v_h.at[pl.ds(base, CHUNK)], vidx_v)
        pltpu.async_copy(h_s.at[uidx_v], hu_v, sem).wait()
        pltpu.async_copy(h_s.at[vidx_v], hv_v, sem).wait()
        for g in range(CHUNK // 16):
            eidx = lax.iota(jnp.int32, 16) + g * 16

            def dbody(db, acc):
                col = jnp.full((16,), db * 16, dtype=jnp.int32)
                for dd in range(16):
                    cc = col + dd
                    acc = acc + (plsc.load_gather(hu_v, [eidx, cc])
                                 * plsc.load_gather(hv_v, [eidx, cc]))
                return acc

            acc = lax.fori_loop(0, D // 16, dbody,
                                jnp.zeros((16,), jnp.float32))
            scr_v[pl.ds(g * 16, 16)] = acc
        pltpu.sync_copy(scr_v, out_h.at[pl.ds(base, CHUNK)])

    nj = 39 + jnp.where(wid < P_CHUNKS - 39 * NW, 1, 0)

    def pbody(j, carry):
        chunk_scores(pu_h, pv_h, pos_h, (wid + j * NW) * CHUNK)
        return carry

    def nbody(j, carry):
        chunk_scores(nu_h, nv_h, neg_h, (wid + j * NW) * CHUNK)
        return carry

    lax.fori_loop(0, nj, pbody, 0)
    lax.fori_loop(0, nj, nbody, 0)


def kernel(feats, edge_index, pos_edge_index, neg_edge_index, W, W_self):
    src = edge_index[0]
    dst = edge_index[1]
    zrow = jnp.zeros((ROWS_PER_TILE, D), jnp.float32)
    zdeg = jnp.zeros((ROWS_PER_TILE, DEGW), jnp.float32)
    ones = jnp.ones((CHUNK, DEGW), jnp.float32)
    aggp, degp = _aggregate(feats, src, dst, zrow, zdeg, ones)
    h = _encode(aggp, degp, feats, W, W_self)
    scores = _score(h, pos_edge_index[0], pos_edge_index[1],
                    neg_edge_index[0], neg_edge_index[1])
    return (scores[0], scores[1])


# speedup vs baseline: 4.8528x; 4.8528x over previous
import functools

import jax
import jax.numpy as jnp
from jax import lax
from jax.experimental import pallas as pl
from jax.experimental.pallas import tpu as pltpu
from jax.experimental.pallas import tpu_sc as plsc

N_NODES = 10000
D = 128
E_BLOCK = 320000
E_PRED = 160000

NC = 2
NS = 16
NW = NC * NS
CHUNK = 128
N_PAD = 10240
ROWS_PER_TILE = N_PAD // NS
DEGW = 16

E_CHUNKS = E_BLOCK // CHUNK
E_BASE = E_CHUNKS // NW
P_CHUNKS = E_PRED // CHUNK
P_BASE = P_CHUNKS // NW

_mesh = plsc.VectorSubcoreMesh(core_axis_name="c", subcore_axis_name="s")


@functools.partial(
    pl.kernel,
    mesh=_mesh,
    out_type=jax.ShapeDtypeStruct((NC, N_PAD, D), jnp.float32),
    scratch_types=[
        pltpu.VMEM_SHARED((N_PAD, D), jnp.float32),
        pltpu.VMEM((CHUNK,), jnp.int32),
        pltpu.VMEM((CHUNK,), jnp.int32),
        pltpu.VMEM((CHUNK, D), jnp.float32),
        pltpu.SemaphoreType.DMA,
    ],
)
def _aggregate(feats_h, src_h, dst_h, zrow_h,
               aggp_h,
               agg_s, sidx_v, didx_v, rows_v, sem):
    c = lax.axis_index("c")
    s = lax.axis_index("s")
    wid = s * NC + c

    r0 = s * ROWS_PER_TILE
    pltpu.sync_copy(zrow_h, rows_v)
    for k in range(ROWS_PER_TILE // CHUNK):
        pltpu.sync_copy(rows_v, agg_s.at[pl.ds(r0 + k * CHUNK, CHUNK)])
    plsc.subcore_barrier()

    nj = E_BASE + jnp.where(wid < E_CHUNKS - E_BASE * NW, 1, 0)

    def body(j, carry):
        base = (wid + j * NW) * CHUNK
        pltpu.sync_copy(src_h.at[pl.ds(base, CHUNK)], sidx_v)
        pltpu.sync_copy(dst_h.at[pl.ds(base, CHUNK)], didx_v)
        pltpu.async_copy(feats_h.at[sidx_v], rows_v, sem).wait()
        pltpu.sync_copy(rows_v, agg_s.at[didx_v], add=True)
        return carry

    lax.fori_loop(0, nj, body, 0)
    plsc.subcore_barrier()

    for k in range(ROWS_PER_TILE // CHUNK):
        rk = r0 + k * CHUNK
        pltpu.sync_copy(agg_s.at[pl.ds(rk, CHUNK)], rows_v)
        pltpu.sync_copy(rows_v, aggp_h.at[c, pl.ds(rk, CHUNK)])


@functools.partial(
    pl.kernel,
    mesh=_mesh,
    out_type=jax.ShapeDtypeStruct((NC, N_PAD, D), jnp.float32),
    scratch_types=[
        pltpu.VMEM_SHARED((N_PAD, D), jnp.float32),
        pltpu.VMEM((CHUNK,), jnp.int32),
        pltpu.VMEM((CHUNK, D), jnp.float32),
        pltpu.SemaphoreType.DMA,
    ],
)
def _degree(dst_h, zrow_h, ones_h,
            degp_h,
            deg_s, didx_v, ones_v, sem):
    c = lax.axis_index("c")
    s = lax.axis_index("s")
    wid = s * NC + c

    r0 = s * ROWS_PER_TILE
    pltpu.sync_copy(zrow_h, ones_v)
    for k in range(ROWS_PER_TILE // CHUNK):
        pltpu.sync_copy(ones_v, deg_s.at[pl.ds(r0 + k * CHUNK, CHUNK)])
    pltpu.sync_copy(ones_h, ones_v)
    plsc.subcore_barrier()

    nj = E_BASE + jnp.where(wid < E_CHUNKS - E_BASE * NW, 1, 0)

    def body(j, carry):
        base = (wid + j * NW) * CHUNK
        pltpu.sync_copy(dst_h.at[pl.ds(base, CHUNK)], didx_v)
        pltpu.sync_copy(ones_v, deg_s.at[didx_v], add=True)
        return carry

    lax.fori_loop(0, nj, body, 0)
    plsc.subcore_barrier()

    for k in range(ROWS_PER_TILE // CHUNK):
        rk = r0 + k * CHUNK
        pltpu.sync_copy(deg_s.at[pl.ds(rk, CHUNK)], ones_v)
        pltpu.sync_copy(ones_v, degp_h.at[c, pl.ds(rk, CHUNK)])


_BR = 1000


def _encode_block(aggp_ref, degp_ref, feats_ref, w_ref, wself_ref, out_ref):
    a = aggp_ref[0] + aggp_ref[1]
    degm = degp_ref[0] + degp_ref[1]
    deg = jnp.maximum(degm[:, 0:1], 1.0)
    norm = a / deg
    h = (jnp.dot(norm, w_ref[...], preferred_element_type=jnp.float32)
         + jnp.dot(feats_ref[...], wself_ref[...],
                   preferred_element_type=jnp.float32))
    out_ref[...] = jnp.maximum(h, 0.0)


def _encode(aggp, degp, feats, W, W_self):
    grid = N_NODES // _BR
    return pl.pallas_call(
        _encode_block,
        grid=(grid,),
        in_specs=[
            pl.BlockSpec((NC, _BR, D), lambda i: (0, i, 0)),
            pl.BlockSpec((NC, _BR, D), lambda i: (0, i, 0)),
            pl.BlockSpec((_BR, D), lambda i: (i, 0)),
            pl.BlockSpec((D, D), lambda i: (0, 0)),
            pl.BlockSpec((D, D), lambda i: (0, 0)),
        ],
        out_specs=pl.BlockSpec((_BR, D), lambda i: (i, 0)),
        out_shape=jax.ShapeDtypeStruct((N_PAD, D), jnp.float32),
    )(aggp, degp, feats, W, W_self)


@functools.partial(
    pl.kernel,
    mesh=_mesh,
    out_type=[
        jax.ShapeDtypeStruct((E_PRED,), jnp.float32),
        jax.ShapeDtypeStruct((E_PRED,), jnp.float32),
    ],
    scratch_types=[
        pltpu.VMEM((CHUNK,), jnp.int32),
        pltpu.VMEM((CHUNK,), jnp.int32),
        pltpu.VMEM((CHUNK, D), jnp.float32),
        pltpu.VMEM((CHUNK, D), jnp.float32),
        pltpu.VMEM((CHUNK,), jnp.float32),
        pltpu.SemaphoreType.DMA,
    ],
    compiler_params=pltpu.CompilerParams(needs_layout_passes=False),
)
def _score(h_h, pu_h, pv_h, nu_h, nv_h,
           pos_h, neg_h,
           uidx_v, vidx_v, hu_v, hv_v, scr_v, sem):
    c = lax.axis_index("c")
    s = lax.axis_index("s")
    wid = s * NC + c

    lane = lax.iota(jnp.int32, 16)

    def chunk_scores(u_h, v_h, out_h, base):
        pltpu.sync_copy(u_h.at[pl.ds(base, CHUNK)], uidx_v)
        pltpu.sync_copy(v_h.at[pl.ds(base, CHUNK)], vidx_v)
        pltpu.async_copy(h_h.at[uidx_v], hu_v, sem).wait()
        pltpu.async_copy(h_h.at[vidx_v], hv_v, sem).wait()
        for g in range(CHUNK // 16):

            def ebody(i, acc):
                e = g * 16 + i
                p = hu_v[e, pl.ds(0, 16)] * hv_v[e, pl.ds(0, 16)]
                for k in range(1, D // 16):
                    p = p + (hu_v[e, pl.ds(k * 16, 16)]
                             * hv_v[e, pl.ds(k * 16, 16)])
                return jnp.where(lane == i, jnp.sum(p), acc)

            acc = lax.fori_loop(0, 16, ebody, jnp.zeros((16,), jnp.float32))
            scr_v[pl.ds(g * 16, 16)] = acc
        pltpu.sync_copy(scr_v, out_h.at[pl.ds(base, CHUNK)])

    nj = P_BASE + jnp.where(wid < P_CHUNKS - P_BASE * NW, 1, 0)

    def pbody(j, carry):
        chunk_scores(pu_h, pv_h, pos_h, (wid + j * NW) * CHUNK)
        return carry

    def nbody(j, carry):
        chunk_scores(nu_h, nv_h, neg_h, (wid + j * NW) * CHUNK)
        return carry

    lax.fori_loop(0, nj, pbody, 0)
    lax.fori_loop(0, nj, nbody, 0)


def kernel(feats, edge_index, pos_edge_index, neg_edge_index, W, W_self):
    src = edge_index[0]
    dst = edge_index[1]
    zrow = jnp.zeros((CHUNK, D), jnp.float32)
    ones = jnp.ones((CHUNK, D), jnp.float32)
    aggp = _aggregate(feats, src, dst, zrow)
    degp = _degree(dst, zrow, ones)
    h = _encode(aggp, degp, feats, W, W_self)
    scores = _score(h, pos_edge_index[0], pos_edge_index[1],
                    neg_edge_index[0], neg_edge_index[1])
    return (scores[0], scores[1])


# speedup vs baseline: 9.7856x; 2.0165x over previous
import functools

import jax
import jax.numpy as jnp
from jax import lax
from jax.experimental import pallas as pl
from jax.experimental.pallas import tpu as pltpu
from jax.experimental.pallas import tpu_sc as plsc

N_NODES = 10000
D = 128
E_BLOCK = 320000
E_PRED = 160000

NC = 2
NS = 16
NW = NC * NS
CHUNK = 128
N_PAD = 10240
ROWS_PER_TILE = N_PAD // NS

E_CHUNKS = E_BLOCK // CHUNK
E_BASE = E_CHUNKS // NW
E_LEFT = E_CHUNKS - E_BASE * NW
P_CHUNKS = E_PRED // CHUNK
P_BASE = P_CHUNKS // NW
P_LEFT = P_CHUNKS - P_BASE * NW

_mesh = plsc.VectorSubcoreMesh(core_axis_name="c", subcore_axis_name="s")


@functools.partial(
    pl.kernel,
    mesh=_mesh,
    out_type=jax.ShapeDtypeStruct((NC, N_PAD, D), jnp.float32),
    scratch_types=[
        pltpu.VMEM_SHARED((N_PAD, D), jnp.float32),
        pltpu.VMEM((E_BASE // 2, 1, CHUNK), jnp.int32),
        pltpu.VMEM((E_BASE // 2, 1, CHUNK), jnp.int32),
        pltpu.VMEM((1, 1, CHUNK), jnp.int32),
        pltpu.VMEM((1, 1, CHUNK), jnp.int32),
        pltpu.VMEM((CHUNK, D), jnp.float32),
        pltpu.VMEM((CHUNK, D), jnp.float32),
        pltpu.SemaphoreType.DMA,
        pltpu.SemaphoreType.DMA,
    ],
)
def _aggregate(feats_h, src2_h, dst2_h, zrow_h,
               aggp_h,
               agg_s, sidx_a, didx_a, sidx_x, didx_x, rows0, rows1,
               sem0, sem1):
    c = lax.axis_index("c")
    s = lax.axis_index("s")
    wid = s * NC + c

    r0 = s * ROWS_PER_TILE
    pltpu.sync_copy(zrow_h, rows0)
    for k in range(ROWS_PER_TILE // CHUNK):
        pltpu.sync_copy(rows0, agg_s.at[pl.ds(r0 + k * CHUNK, CHUNK)])
    plsc.subcore_barrier()

    def gather(j, rows_v, sem):
        return pltpu.async_copy(feats_h.at[sidx_a.at[j, 0]], rows_v, sem)

    def gwait(j, rows_v, sem):
        pltpu.make_async_copy(feats_h.at[sidx_a.at[j, 0]], rows_v,
                              sem).wait()

    def scat(j, rows_v):
        pltpu.sync_copy(rows_v, agg_s.at[didx_a.at[j, 0]], add=True)

    STG = E_BASE // 2
    for t in range(2):
        cb = wid * E_BASE + t * STG
        pltpu.sync_copy(src2_h.at[pl.ds(cb, STG)], sidx_a)
        pltpu.sync_copy(dst2_h.at[pl.ds(cb, STG)], didx_a)
        gather(0, rows0, sem0)

        def pair(i, carry):
            j0 = 2 * i
            gather(j0 + 1, rows1, sem1)
            gwait(j0, rows0, sem0)
            scat(j0, rows0)
            gather(j0 + 2, rows0, sem0)
            gwait(j0 + 1, rows1, sem1)
            scat(j0 + 1, rows1)
            return carry

        lax.fori_loop(0, (STG - 1) // 2, pair, 0)
        gwait(STG - 1, rows0, sem0)
        scat(STG - 1, rows0)

    def extra(j, carry):
        xb = NW * E_BASE + wid
        pltpu.sync_copy(src2_h.at[pl.ds(xb, 1)], sidx_x)
        pltpu.sync_copy(dst2_h.at[pl.ds(xb, 1)], didx_x)
        pltpu.async_copy(feats_h.at[sidx_x.at[0, 0]], rows0, sem0).wait()
        pltpu.sync_copy(rows0, agg_s.at[didx_x.at[0, 0]], add=True)
        return carry

    lax.fori_loop(0, jnp.where(wid < E_LEFT, 1, 0), extra, 0)
    plsc.subcore_barrier()

    for k in range(ROWS_PER_TILE // CHUNK):
        rk = r0 + k * CHUNK
        pltpu.sync_copy(agg_s.at[pl.ds(rk, CHUNK)], rows0)
        pltpu.sync_copy(rows0, aggp_h.at[c, pl.ds(rk, CHUNK)])


@functools.partial(
    pl.kernel,
    mesh=_mesh,
    out_type=jax.ShapeDtypeStruct((NC, N_PAD, D), jnp.float32),
    scratch_types=[
        pltpu.VMEM_SHARED((N_PAD, D), jnp.float32),
        pltpu.VMEM((E_BASE, 1, CHUNK), jnp.int32),
        pltpu.VMEM((1, 1, CHUNK), jnp.int32),
        pltpu.VMEM((CHUNK, D), jnp.float32),
        pltpu.SemaphoreType.DMA,
    ],
)
def _degree(dst2_h, zrow_h, ones_h,
            degp_h,
            deg_s, didx_a, didx_x, ones_v, sem):
    c = lax.axis_index("c")
    s = lax.axis_index("s")
    wid = s * NC + c

    r0 = s * ROWS_PER_TILE
    pltpu.sync_copy(zrow_h, ones_v)
    for k in range(ROWS_PER_TILE // CHUNK):
        pltpu.sync_copy(ones_v, deg_s.at[pl.ds(r0 + k * CHUNK, CHUNK)])
    cb = wid * E_BASE
    pltpu.sync_copy(dst2_h.at[pl.ds(cb, E_BASE)], didx_a)
    pltpu.sync_copy(ones_h, ones_v)
    plsc.subcore_barrier()

    pltpu.async_copy(ones_v, deg_s.at[didx_a.at[0, 0]], sem, add=True)

    def body(j, carry):
        pltpu.async_copy(ones_v, deg_s.at[didx_a.at[j, 0]], sem, add=True)
        pltpu.make_async_copy(ones_v, deg_s.at[didx_a.at[j - 1, 0]],
                              sem).wait()
        return carry

    lax.fori_loop(1, E_BASE, body, 0)
    pltpu.make_async_copy(ones_v, deg_s.at[didx_a.at[E_BASE - 1, 0]],
                          sem).wait()

    def extra(j, carry):
        xb = NW * E_BASE + wid
        pltpu.sync_copy(dst2_h.at[pl.ds(xb, 1)], didx_x)
        pltpu.sync_copy(ones_v, deg_s.at[didx_x.at[0, 0]], add=True)
        return carry

    lax.fori_loop(0, jnp.where(wid < E_LEFT, 1, 0), extra, 0)
    plsc.subcore_barrier()

    for k in range(ROWS_PER_TILE // CHUNK):
        rk = r0 + k * CHUNK
        pltpu.sync_copy(deg_s.at[pl.ds(rk, CHUNK)], ones_v)
        pltpu.sync_copy(ones_v, degp_h.at[c, pl.ds(rk, CHUNK)])


_BR = 1000


def _encode_block(aggp_ref, degp_ref, feats_ref, w_ref, wself_ref, out_ref):
    a = aggp_ref[0] + aggp_ref[1]
    degm = degp_ref[0] + degp_ref[1]
    deg = jnp.maximum(degm[:, 0:1], 1.0)
    norm = a / deg
    h = (jnp.dot(norm, w_ref[...], preferred_element_type=jnp.float32)
         + jnp.dot(feats_ref[...], wself_ref[...],
                   preferred_element_type=jnp.float32))
    out_ref[...] = jnp.maximum(h, 0.0)


def _encode(aggp, degp, feats, W, W_self):
    grid = N_NODES // _BR
    return pl.pallas_call(
        _encode_block,
        grid=(grid,),
        in_specs=[
            pl.BlockSpec((NC, _BR, D), lambda i: (0, i, 0)),
            pl.BlockSpec((NC, _BR, D), lambda i: (0, i, 0)),
            pl.BlockSpec((_BR, D), lambda i: (i, 0)),
            pl.BlockSpec((D, D), lambda i: (0, 0)),
            pl.BlockSpec((D, D), lambda i: (0, 0)),
        ],
        out_specs=pl.BlockSpec((_BR, D), lambda i: (i, 0)),
        out_shape=jax.ShapeDtypeStruct((N_PAD, D), jnp.float32),
    )(aggp, degp, feats, W, W_self)


@functools.partial(
    pl.kernel,
    mesh=_mesh,
    out_type=[
        jax.ShapeDtypeStruct((P_CHUNKS, 1, CHUNK), jnp.float32),
        jax.ShapeDtypeStruct((P_CHUNKS, 1, CHUNK), jnp.float32),
    ],
    scratch_types=[
        pltpu.VMEM((P_BASE, 1, CHUNK), jnp.int32),
        pltpu.VMEM((P_BASE, 1, CHUNK), jnp.int32),
        pltpu.VMEM((P_BASE, 1, CHUNK), jnp.int32),
        pltpu.VMEM((P_BASE, 1, CHUNK), jnp.int32),
        pltpu.VMEM((1, 1, CHUNK), jnp.int32),
        pltpu.VMEM((1, 1, CHUNK), jnp.int32),
        pltpu.VMEM((CHUNK, D), jnp.float32),
        pltpu.VMEM((CHUNK, D), jnp.float32),
        pltpu.VMEM((CHUNK, D), jnp.float32),
        pltpu.VMEM((CHUNK, D), jnp.float32),
        pltpu.VMEM((P_BASE, 1, CHUNK), jnp.float32),
        pltpu.VMEM((1, 1, CHUNK), jnp.float32),
        pltpu.SemaphoreType.DMA,
        pltpu.SemaphoreType.DMA,
    ],
    compiler_params=pltpu.CompilerParams(needs_layout_passes=False),
)
def _score(h_h, pu2_h, pv2_h, nu2_h, nv2_h,
           pos2_h, neg2_h,
           uidx_a, vidx_a, uidx_b, vidx_b, uidx_x, vidx_x,
           hu0, hv0, hu1, hv1, sall_v, sx_v, sem0, sem1):
    c = lax.axis_index("c")
    s = lax.axis_index("s")
    wid = s * NC + c
    cb = wid * P_BASE

    pltpu.sync_copy(pu2_h.at[pl.ds(cb, P_BASE)], uidx_a)
    pltpu.sync_copy(pv2_h.at[pl.ds(cb, P_BASE)], vidx_a)
    pltpu.sync_copy(nu2_h.at[pl.ds(cb, P_BASE)], uidx_b)
    pltpu.sync_copy(nv2_h.at[pl.ds(cb, P_BASE)], vidx_b)

    lane = lax.iota(jnp.int32, 16)

    def compute(hu_v, hv_v, out_ref, j):
        for g in range(CHUNK // 16):

            def ebody(i, acc):
                e = g * 16 + i
                p = hu_v[e, pl.ds(0, 16)] * hv_v[e, pl.ds(0, 16)]
                for k in range(1, D // 16):
                    p = p + (hu_v[e, pl.ds(k * 16, 16)]
                             * hv_v[e, pl.ds(k * 16, 16)])
                return jnp.where(lane == i, jnp.sum(p), acc)

            acc = lax.fori_loop(0, 16, ebody, jnp.zeros((16,), jnp.float32))
            out_ref[j, 0, pl.ds(g * 16, 16)] = acc

    def run(u_a, v_a, out_h):
        def fire(j, hu_v, hv_v, sem):
            pltpu.async_copy(h_h.at[u_a.at[j, 0]], hu_v, sem)
            pltpu.async_copy(h_h.at[v_a.at[j, 0]], hv_v, sem)

        def wait(j, hu_v, hv_v, sem):
            pltpu.make_async_copy(h_h.at[u_a.at[j, 0]], hu_v, sem).wait()
            pltpu.make_async_copy(h_h.at[v_a.at[j, 0]], hv_v, sem).wait()

        fire(0, hu0, hv0, sem0)

        def pair(i, carry):
            j0 = 2 * i
            fire(j0 + 1, hu1, hv1, sem1)
            wait(j0, hu0, hv0, sem0)
            compute(hu0, hv0, sall_v, j0)
            fire(j0 + 2, hu0, hv0, sem0)
            wait(j0 + 1, hu1, hv1, sem1)
            compute(hu1, hv1, sall_v, j0 + 1)
            return carry

        lax.fori_loop(0, (P_BASE - 1) // 2, pair, 0)
        j0 = P_BASE - 1
        wait(j0, hu0, hv0, sem0)
        compute(hu0, hv0, sall_v, j0)
        pltpu.sync_copy(sall_v, out_h.at[pl.ds(cb, P_BASE)])

    run(uidx_a, vidx_a, pos2_h)
    run(uidx_b, vidx_b, neg2_h)

    def extra_one(u2_h, v2_h, out_h):
        xb = NW * P_BASE + wid
        pltpu.sync_copy(u2_h.at[pl.ds(xb, 1)], uidx_x)
        pltpu.sync_copy(v2_h.at[pl.ds(xb, 1)], vidx_x)
        pltpu.async_copy(h_h.at[uidx_x.at[0, 0]], hu0, sem0).wait()
        pltpu.async_copy(h_h.at[vidx_x.at[0, 0]], hv0, sem0).wait()
        compute(hu0, hv0, sx_v, 0)
        pltpu.sync_copy(sx_v, out_h.at[pl.ds(xb, 1)])

    def extra(j, carry):
        extra_one(pu2_h, pv2_h, pos2_h)
        extra_one(nu2_h, nv2_h, neg2_h)
        return carry

    lax.fori_loop(0, jnp.where(wid < P_LEFT, 1, 0), extra, 0)


def kernel(feats, edge_index, pos_edge_index, neg_edge_index, W, W_self):
    src2 = edge_index[0].reshape(E_CHUNKS, 1, CHUNK)
    dst2 = edge_index[1].reshape(E_CHUNKS, 1, CHUNK)
    zrow = jnp.zeros((CHUNK, D), jnp.float32)
    ones = jnp.ones((CHUNK, D), jnp.float32)
    aggp = _aggregate(feats, src2, dst2, zrow)
    degp = _degree(dst2, zrow, ones)
    h = _encode(aggp, degp, feats, W, W_self)
    scores = _score(h,
                    pos_edge_index[0].reshape(P_CHUNKS, 1, CHUNK),
                    pos_edge_index[1].reshape(P_CHUNKS, 1, CHUNK),
                    neg_edge_index[0].reshape(P_CHUNKS, 1, CHUNK),
                    neg_edge_index[1].reshape(P_CHUNKS, 1, CHUNK))
    return (scores[0].reshape(E_PRED), scores[1].reshape(E_PRED))
